# Initial kernel scaffold; baseline (speedup 1.0000x reference)
import jax
import jax.numpy as jnp
from jax.experimental import pallas as pl


def kernel(norm_adj, user_emb, item_emb):
    raise NotImplementedError("write your pallas kernel here")


# speedup vs baseline: 1.0718x; 1.0718x over previous
import jax
import jax.numpy as jnp
from jax.experimental import pallas as pl
from jax.experimental.pallas import tpu as pltpu

_N_USERS = 8192
_N_ITEMS = 8192
_EMB = 16
_N = _N_USERS + _N_ITEMS

_BM1 = 256
_BM2 = 512


def _pass1_body(a_ref, e0_ref, e1_ref, abf_ref):
    a_bf = a_ref[...].astype(jnp.bfloat16)
    abf_ref[...] = a_bf
    e1_ref[...] = jnp.dot(a_bf, e0_ref[...], preferred_element_type=jnp.float32)


def _pass2_body(a_ref, e_ref, out_ref):
    out_ref[...] = jnp.dot(a_ref[...], e_ref[...], preferred_element_type=jnp.float32)


def _pass3_body(a_ref, e2b_ref, e0_ref, e1_ref, e2_ref, out_ref):
    e3 = jnp.dot(a_ref[...], e2b_ref[...], preferred_element_type=jnp.float32)
    out_ref[...] = 0.25 * (e0_ref[...] + e1_ref[...] + e2_ref[...] + e3)


def kernel(norm_adj, user_emb, item_emb):
    e0 = jnp.concatenate([user_emb, item_emb], axis=0)
    e0_bf = e0.astype(jnp.bfloat16)

    e1, a_bf = pl.pallas_call(
        _pass1_body,
        grid=(_N // _BM1,),
        in_specs=[
            pl.BlockSpec((_BM1, _N), lambda i: (i, 0)),
            pl.BlockSpec((_N, _EMB), lambda i: (0, 0)),
        ],
        out_specs=[
            pl.BlockSpec((_BM1, _EMB), lambda i: (i, 0)),
            pl.BlockSpec((_BM1, _N), lambda i: (i, 0)),
        ],
        out_shape=[
            jax.ShapeDtypeStruct((_N, _EMB), jnp.float32),
            jax.ShapeDtypeStruct((_N, _N), jnp.bfloat16),
        ],
        compiler_params=pltpu.CompilerParams(
            dimension_semantics=("arbitrary",),
        ),
    )(norm_adj, e0_bf)

    e2 = pl.pallas_call(
        _pass2_body,
        grid=(_N // _BM2,),
        in_specs=[
            pl.BlockSpec((_BM2, _N), lambda i: (i, 0)),
            pl.BlockSpec((_N, _EMB), lambda i: (0, 0)),
        ],
        out_specs=pl.BlockSpec((_BM2, _EMB), lambda i: (i, 0)),
        out_shape=jax.ShapeDtypeStruct((_N, _EMB), jnp.float32),
        compiler_params=pltpu.CompilerParams(
            dimension_semantics=("arbitrary",),
        ),
    )(a_bf, e1.astype(jnp.bfloat16))

    final = pl.pallas_call(
        _pass3_body,
        grid=(_N // _BM2,),
        in_specs=[
            pl.BlockSpec((_BM2, _N), lambda i: (i, 0)),
            pl.BlockSpec((_N, _EMB), lambda i: (0, 0)),
            pl.BlockSpec((_BM2, _EMB), lambda i: (i, 0)),
            pl.BlockSpec((_BM2, _EMB), lambda i: (i, 0)),
            pl.BlockSpec((_BM2, _EMB), lambda i: (i, 0)),
        ],
        out_specs=pl.BlockSpec((_BM2, _EMB), lambda i: (i, 0)),
        out_shape=jax.ShapeDtypeStruct((_N, _EMB), jnp.float32),
        compiler_params=pltpu.CompilerParams(
            dimension_semantics=("arbitrary",),
        ),
    )(a_bf, e2.astype(jnp.bfloat16), e0, e1, e2)

    return (final[:_N_USERS], final[_N_USERS:])


# speedup vs baseline: 1.0910x; 1.0179x over previous
import jax
import jax.numpy as jnp
from jax.experimental import pallas as pl
from jax.experimental.pallas import tpu as pltpu

_N_USERS = 8192
_N_ITEMS = 8192
_EMB = 16
_N = _N_USERS + _N_ITEMS

_BM1 = 256
_BM2 = 512


def _pass1_body(a_ref, e0_ref, e1f_ref, e1b_ref, abf_ref):
    a_bf = a_ref[...].astype(jnp.bfloat16)
    abf_ref[...] = a_bf
    e1 = jnp.dot(a_bf, e0_ref[...], preferred_element_type=jnp.float32)
    e1f_ref[...] = e1
    e1b_ref[...] = e1.astype(jnp.bfloat16)


def _pass23_body(a_ref, e1b_ref, e0_ref, e1_ref, out_ref, e2f_ref, e2b_ref):
    p = pl.program_id(0)
    i = pl.program_id(1)

    @pl.when(p == 0)
    def _phase2():
        e2 = jnp.dot(a_ref[...], e1b_ref[...], preferred_element_type=jnp.float32)
        e2f_ref[pl.ds(i * _BM2, _BM2), :] = e2
        e2b_ref[pl.ds(i * _BM2, _BM2), :] = e2.astype(jnp.bfloat16)
        out_ref[...] = e2

    @pl.when(p == 1)
    def _phase3():
        e3 = jnp.dot(a_ref[...], e2b_ref[...], preferred_element_type=jnp.float32)
        e2 = e2f_ref[pl.ds(i * _BM2, _BM2), :]
        out_ref[...] = 0.25 * (e0_ref[...] + e1_ref[...] + e2 + e3)


def kernel(norm_adj, user_emb, item_emb):
    e0 = jnp.concatenate([user_emb, item_emb], axis=0)
    e0_bf = e0.astype(jnp.bfloat16)

    e1, e1_bf, a_bf = pl.pallas_call(
        _pass1_body,
        grid=(_N // _BM1,),
        in_specs=[
            pl.BlockSpec((_BM1, _N), lambda i: (i, 0)),
            pl.BlockSpec((_N, _EMB), lambda i: (0, 0)),
        ],
        out_specs=[
            pl.BlockSpec((_BM1, _EMB), lambda i: (i, 0)),
            pl.BlockSpec((_BM1, _EMB), lambda i: (i, 0)),
            pl.BlockSpec((_BM1, _N), lambda i: (i, 0)),
        ],
        out_shape=[
            jax.ShapeDtypeStruct((_N, _EMB), jnp.float32),
            jax.ShapeDtypeStruct((_N, _EMB), jnp.bfloat16),
            jax.ShapeDtypeStruct((_N, _N), jnp.bfloat16),
        ],
        compiler_params=pltpu.CompilerParams(
            dimension_semantics=("arbitrary",),
        ),
    )(norm_adj, e0_bf)

    final = pl.pallas_call(
        _pass23_body,
        grid=(2, _N // _BM2),
        in_specs=[
            pl.BlockSpec((_BM2, _N), lambda p, i: (i, 0)),
            pl.BlockSpec((_N, _EMB), lambda p, i: (0, 0)),
            pl.BlockSpec((_BM2, _EMB), lambda p, i: (i, 0)),
            pl.BlockSpec((_BM2, _EMB), lambda p, i: (i, 0)),
        ],
        out_specs=pl.BlockSpec((_BM2, _EMB), lambda p, i: (i, 0)),
        out_shape=jax.ShapeDtypeStruct((_N, _EMB), jnp.float32),
        scratch_shapes=[
            pltpu.VMEM((_N, _EMB), jnp.float32),
            pltpu.VMEM((_N, _EMB), jnp.bfloat16),
        ],
        compiler_params=pltpu.CompilerParams(
            dimension_semantics=("arbitrary", "arbitrary"),
        ),
    )(a_bf, e1_bf, e0, e1)

    return (final[:_N_USERS], final[_N_USERS:])
